# Initial kernel scaffold; baseline (speedup 1.0000x reference)
import jax
import jax.numpy as jnp
from jax import lax
from jax.experimental import pallas as pl
from jax.experimental.pallas import tpu as pltpu
from jax.experimental.pallas import tpu_sc as plsc

B, H, S, D = 8, 8, 8192, 128
K = max(1, min(S, int(S * (1.0 - 0.8))))
SBLK = 1024
LANES = 16
NCHUNK = S // LANES
GCH = 126
NG = K // GCH
IDX_PAD = ((K + LANES - 1) // LANES) * LANES


def _norms_body(keys_ref, out_ref):
    x = keys_ref[0]
    ss = jnp.sum(x * x, axis=-1)
    out_ref[0] = jnp.mean(jnp.sqrt(ss), axis=0)


def _norms(keys):
    return pl.pallas_call(
        _norms_body,
        grid=(B, S // SBLK),
        in_specs=[pl.BlockSpec((1, H, SBLK, D), lambda b, s: (b, 0, s, 0))],
        out_specs=pl.BlockSpec((1, SBLK), lambda b, s: (b, s)),
        out_shape=jax.ShapeDtypeStruct((B, S), jnp.float32),
    )(keys)


def _select_body(scores_ref, thr_ref, quota_ref):
    bits = lax.bitcast_convert_type(scores_ref[...], jnp.int32)

    def body(_, carry):
        lo, hi = carry
        mid = lo + ((hi - lo + 1) >> 1)
        cnt = jnp.sum((bits >= mid).astype(jnp.int32), axis=1, keepdims=True)
        ge = cnt >= K
        return jnp.where(ge, mid, lo), jnp.where(ge, hi, mid - 1)

    lo0 = jnp.zeros((B, 1), jnp.int32)
    hi0 = jnp.full((B, 1), 0x7F800000, jnp.int32)
    lo, _ = lax.fori_loop(0, 31, body, (lo0, hi0))
    cnt_gt = jnp.sum((bits >= (lo + 1)).astype(jnp.int32), axis=1,
                     keepdims=True)
    quota = K - cnt_gt
    thr_ref[...] = jnp.broadcast_to(
        lax.bitcast_convert_type(lo, jnp.float32), (B, 128))
    quota_ref[...] = jnp.broadcast_to(quota, (B, 128))


def _select(scores):
    return pl.pallas_call(
        _select_body,
        grid=(1,),
        in_specs=[pl.BlockSpec((B, S), lambda i: (0, 0))],
        out_specs=(pl.BlockSpec((B, 128), lambda i: (0, 0)),
                   pl.BlockSpec((B, 128), lambda i: (0, 0))),
        out_shape=(jax.ShapeDtypeStruct((B, 128), jnp.float32),
                   jax.ShapeDtypeStruct((B, 128), jnp.int32)),
    )(scores)


def _sc_body(keys_hbm, values_hbm, scores_hbm, thr_hbm, quota_hbm,
             outk_hbm, outv_hbm,
             scores_v, thr_v, quota_v, idx_v, gidx_v, buf_v, sem):
    c = lax.axis_index("core")
    sub = lax.axis_index("subcore")
    wid = sub * 2 + c
    b = wid // 4
    r = wid % 4

    pltpu.sync_copy(scores_hbm.at[b], scores_v)
    pltpu.sync_copy(thr_hbm.at[b], thr_v)
    pltpu.sync_copy(quota_hbm.at[b], quota_v)
    tvec = thr_v[pl.ds(0, LANES)]
    qvec = quota_v[pl.ds(0, LANES)]

    def chunk(ci, carry):
        off, ecnt = carry
        sv = scores_v[pl.ds(ci * LANES, LANES)]
        gt = sv > tvec
        eq = sv == tvec
        eq_i = jnp.where(eq, 1, 0).astype(jnp.int32)
        eq_rank = (ecnt + plsc.cumsum(eq_i)) - eq_i
        keep = gt | (eq & (eq_rank < qvec))
        keep_i = jnp.where(keep, 1, 0).astype(jnp.int32)
        pos = (off + plsc.cumsum(keep_i)) - 1
        idxvec = ci * LANES + lax.iota(jnp.int32, LANES)
        plsc.store_scatter(idx_v, [pos], idxvec, mask=keep)
        return off + jnp.sum(keep_i), ecnt + jnp.sum(eq_i)

    lax.fori_loop(0, NCHUNK, chunk, (jnp.int32(0), jnp.int32(0)))

    for j in range(2):
        h = r * 2 + j
        bh = b * 8 + h
        row0 = bh * S

        @pl.loop(0, IDX_PAD // LANES)
        def _(ci):
            sl = pl.ds(ci * LANES, LANES)
            gidx_v[sl] = idx_v[sl] + row0

        for tbl, out in ((keys_hbm, outk_hbm), (values_hbm, outv_hbm)):
            @pl.loop(0, NG)
            def _(ci):
                base = ci * GCH
                pltpu.async_copy(
                    tbl.at[gidx_v.at[pl.ds(base, GCH)]], buf_v, sem).wait()
                pltpu.sync_copy(buf_v, out.at[pl.ds(bh * K + base, GCH)])


def _sc_gather(keys2d, values2d, scores, thr, quota):
    mesh = plsc.VectorSubcoreMesh(core_axis_name="core",
                                  subcore_axis_name="subcore")
    kern = pl.kernel(
        _sc_body,
        out_type=(jax.ShapeDtypeStruct((B * H * K, D), jnp.float32),
                  jax.ShapeDtypeStruct((B * H * K, D), jnp.float32)),
        mesh=mesh,
        scratch_types=[
            pltpu.VMEM((S,), jnp.float32),
            pltpu.VMEM((128,), jnp.float32),
            pltpu.VMEM((128,), jnp.int32),
            pltpu.VMEM((IDX_PAD,), jnp.int32),
            pltpu.VMEM((IDX_PAD,), jnp.int32),
            pltpu.VMEM((GCH, D), jnp.float32),
            pltpu.SemaphoreType.DMA,
        ],
    )
    return kern(keys2d, values2d, scores, thr, quota)


@jax.jit
def kernel(keys, values):
    scores = _norms(keys)
    thr, quota = _select(scores)
    keys2d = keys.reshape(B * H * S, D)
    values2d = values.reshape(B * H * S, D)
    outk, outv = _sc_gather(keys2d, values2d, scores, thr, quota)
    return outk.reshape(B, H, K, D), outv.reshape(B, H, K, D)


# speedup vs baseline: 5.2916x; 5.2916x over previous
import dataclasses

import jax
import jax.numpy as jnp
from jax import lax
from jax.experimental import pallas as pl
from jax.experimental.pallas import tpu as pltpu
from jax.experimental.pallas import tpu_sc as plsc

B, H, S, D = 8, 8, 8192, 128
K = max(1, min(S, int(S * (1.0 - 0.8))))
SBLK = 1024
LANES = 16
NCHUNK = S // LANES
GCH = 128
NGF = K // GCH
REM = K - NGF * GCH
IDX_PAD = (NGF + 1) * GCH


def _norms_body(keys_ref, out_ref):
    x = keys_ref[0]
    ss = jnp.sum(x * x, axis=-1)
    out_ref[0, 0] = jnp.mean(jnp.sqrt(ss), axis=0)


def _norms(keys):
    out = pl.pallas_call(
        _norms_body,
        grid=(B, S // SBLK),
        in_specs=[pl.BlockSpec((1, H, SBLK, D), lambda b, s: (b, 0, s, 0))],
        out_specs=pl.BlockSpec((1, 1, SBLK), lambda b, s: (b, 0, s)),
        out_shape=jax.ShapeDtypeStruct((B, 1, S), jnp.float32),
    )(keys)
    return out


def _select_body(scores_ref, thr_ref, quota_ref):
    bits = lax.bitcast_convert_type(scores_ref[:, 0, :], jnp.int32)

    def body(_, carry):
        lo, hi = carry
        mid = lo + ((hi - lo + 1) >> 1)
        cnt = jnp.sum((bits >= mid).astype(jnp.int32), axis=1, keepdims=True)
        ge = cnt >= K
        return jnp.where(ge, mid, lo), jnp.where(ge, hi, mid - 1)

    lo0 = jnp.zeros((B, 1), jnp.int32)
    hi0 = jnp.full((B, 1), 0x7F800000, jnp.int32)
    lo, _ = lax.fori_loop(0, 31, body, (lo0, hi0))
    cnt_gt = jnp.sum((bits >= (lo + 1)).astype(jnp.int32), axis=1,
                     keepdims=True)
    quota = K - cnt_gt
    thr_ref[...] = jnp.broadcast_to(
        lax.bitcast_convert_type(lo, jnp.float32), (B, 128))
    quota_ref[...] = jnp.broadcast_to(quota, (B, 128))


def _select(scores):
    return pl.pallas_call(
        _select_body,
        grid=(1,),
        in_specs=[pl.BlockSpec((B, 1, S), lambda i: (0, 0, 0))],
        out_specs=(pl.BlockSpec((B, 128), lambda i: (0, 0)),
                   pl.BlockSpec((B, 128), lambda i: (0, 0))),
        out_shape=(jax.ShapeDtypeStruct((B, 128), jnp.float32),
                   jax.ShapeDtypeStruct((B, 128), jnp.int32)),
    )(scores)


def _sc_body(keys_hbm, values_hbm, scores_hbm, thr_hbm, quota_hbm,
             outk_hbm, outv_hbm,
             scores_v, thr_v, quota_v, idx_v, gidx_v, buf_v, sem):
    c = lax.axis_index("core")
    sub = lax.axis_index("subcore")
    wid = sub * 2 + c
    b = wid // 4
    r = wid % 4

    pltpu.sync_copy(scores_hbm.at[b, 0], scores_v)
    pltpu.sync_copy(thr_hbm.at[b, 0], thr_v)
    pltpu.sync_copy(quota_hbm.at[b, 0], quota_v)
    tvec = thr_v[pl.ds(0, LANES)]
    qvec = quota_v[pl.ds(0, LANES)]

    zeros16 = jnp.zeros((LANES,), jnp.int32)
    idx_v[pl.ds(IDX_PAD - 2 * LANES, LANES)] = zeros16
    idx_v[pl.ds(IDX_PAD - LANES, LANES)] = zeros16

    def chunk(ci, carry):
        off, ecnt = carry
        sv = scores_v[pl.ds(ci * LANES, LANES)]
        gt = sv > tvec
        eq = sv == tvec
        eq_i = jnp.where(eq, 1, 0).astype(jnp.int32)
        eq_rank = (ecnt + plsc.cumsum(eq_i)) - eq_i
        keep = gt | (eq & (eq_rank < qvec))
        keep_i = jnp.where(keep, 1, 0).astype(jnp.int32)
        pos = (off + plsc.cumsum(keep_i)) - 1
        idxvec = ci * LANES + lax.iota(jnp.int32, LANES)
        plsc.store_scatter(idx_v, [pos], idxvec, mask=keep)
        return off + jnp.sum(keep_i), ecnt + jnp.sum(eq_i)

    lax.fori_loop(0, NCHUNK, chunk, (jnp.int32(0), jnp.int32(0)))

    for j in range(2):
        h = r * 2 + j
        bh = b * 8 + h
        row0 = bh * S

        @pl.loop(0, IDX_PAD // LANES)
        def _(ci):
            sl = pl.ds(ci * LANES, LANES)
            gidx_v[sl] = idx_v[sl] + row0

        for tbl, out in ((keys_hbm, outk_hbm), (values_hbm, outv_hbm)):
            @pl.loop(0, NGF)
            def _(ci):
                base = ci * GCH
                pltpu.async_copy(
                    tbl.at[gidx_v.at[pl.ds(base, GCH)]], buf_v, sem).wait()
                pltpu.sync_copy(buf_v, out.at[bh, pl.ds(base, GCH)])

            pltpu.async_copy(
                tbl.at[gidx_v.at[pl.ds(NGF * GCH, GCH)]], buf_v, sem).wait()
            pltpu.sync_copy(buf_v.at[pl.ds(0, REM)],
                            out.at[bh, pl.ds(NGF * GCH, REM)])


def _sc_gather(keys2d, values2d, scores, thr, quota):
    mesh = plsc.VectorSubcoreMesh(core_axis_name="core",
                                  subcore_axis_name="subcore")
    cp = pltpu.CompilerParams()
    if "needs_layout_passes" in pltpu.CompilerParams.__dataclass_fields__:
        cp = dataclasses.replace(cp, needs_layout_passes=False)
    kern = pl.kernel(
        _sc_body,
        compiler_params=cp,
        out_type=(jax.ShapeDtypeStruct((B * H, K, D), jnp.float32),
                  jax.ShapeDtypeStruct((B * H, K, D), jnp.float32)),
        mesh=mesh,
        scratch_types=[
            pltpu.VMEM((S,), jnp.float32),
            pltpu.VMEM((128,), jnp.float32),
            pltpu.VMEM((128,), jnp.int32),
            pltpu.VMEM((IDX_PAD,), jnp.int32),
            pltpu.VMEM((IDX_PAD,), jnp.int32),
            pltpu.VMEM((GCH, D), jnp.float32),
            pltpu.SemaphoreType.DMA,
        ],
    )
    return kern(keys2d, values2d, scores, thr, quota)


@jax.jit
def kernel(keys, values):
    scores3 = _norms(keys)
    thr, quota = _select(scores3)
    keys2d = keys.reshape(B * H * S, D)
    values2d = values.reshape(B * H * S, D)
    outk, outv = _sc_gather(keys2d, values2d, scores3,
                            thr.reshape(B, 1, 128), quota.reshape(B, 1, 128))
    return outk.reshape(B, H, K, D), outv.reshape(B, H, K, D)


# speedup vs baseline: 5.6738x; 1.0722x over previous
import dataclasses

import jax
import jax.numpy as jnp
from jax import lax
from jax.experimental import pallas as pl
from jax.experimental.pallas import tpu as pltpu
from jax.experimental.pallas import tpu_sc as plsc

B, H, S, D = 8, 8, 8192, 128
K = max(1, min(S, int(S * (1.0 - 0.8))))
SBLK = 1024
LANES = 16
NCHUNK = S // LANES
GCH = 128
NGF = K // GCH
REM = K - NGF * GCH
IDX_PAD = (NGF + 1) * GCH


def _norms_body(keys_ref, out_ref):
    x = keys_ref[0]
    ss = jnp.sum(x * x, axis=-1)
    out_ref[0, 0] = jnp.mean(jnp.sqrt(ss), axis=0)


def _norms(keys):
    out = pl.pallas_call(
        _norms_body,
        grid=(B, S // SBLK),
        in_specs=[pl.BlockSpec((1, H, SBLK, D), lambda b, s: (b, 0, s, 0))],
        out_specs=pl.BlockSpec((1, 1, SBLK), lambda b, s: (b, 0, s)),
        out_shape=jax.ShapeDtypeStruct((B, 1, S), jnp.float32),
    )(keys)
    return out


def _select_body(scores_ref, thr_ref, quota_ref):
    bits = lax.bitcast_convert_type(scores_ref[:, 0, :], jnp.int32)

    def body(_, carry):
        lo, hi = carry
        mid = lo + ((hi - lo + 1) >> 1)
        cnt = jnp.sum((bits >= mid).astype(jnp.int32), axis=1, keepdims=True)
        ge = cnt >= K
        return jnp.where(ge, mid, lo), jnp.where(ge, hi, mid - 1)

    lo0 = jnp.zeros((B, 1), jnp.int32)
    hi0 = jnp.full((B, 1), 0x7F800000, jnp.int32)
    lo, _ = lax.fori_loop(0, 31, body, (lo0, hi0))
    cnt_gt = jnp.sum((bits >= (lo + 1)).astype(jnp.int32), axis=1,
                     keepdims=True)
    quota = K - cnt_gt
    thr_ref[...] = jnp.broadcast_to(
        lax.bitcast_convert_type(lo, jnp.float32), (B, 128))
    quota_ref[...] = jnp.broadcast_to(quota, (B, 128))


def _select(scores):
    return pl.pallas_call(
        _select_body,
        grid=(1,),
        in_specs=[pl.BlockSpec((B, 1, S), lambda i: (0, 0, 0))],
        out_specs=(pl.BlockSpec((B, 128), lambda i: (0, 0)),
                   pl.BlockSpec((B, 128), lambda i: (0, 0))),
        out_shape=(jax.ShapeDtypeStruct((B, 128), jnp.float32),
                   jax.ShapeDtypeStruct((B, 128), jnp.int32)),
    )(scores)


def _sc_body(keys_hbm, values_hbm, scores_hbm, thr_hbm, quota_hbm,
             outk_hbm, outv_hbm,
             scores_v, thr_v, quota_v, idx_v, bufa_v, bufb_v,
             gsa, gsb, wsa, wsb):
    c = lax.axis_index("core")
    sub = lax.axis_index("subcore")
    wid = sub * 2 + c
    b = wid // 4
    r = wid % 4

    pltpu.sync_copy(scores_hbm.at[b, 0], scores_v)
    pltpu.sync_copy(thr_hbm.at[b, 0], thr_v)
    pltpu.sync_copy(quota_hbm.at[b, 0], quota_v)
    tvec = thr_v[pl.ds(0, LANES)]
    qvec = quota_v[pl.ds(0, LANES)]

    zeros16 = jnp.zeros((LANES,), jnp.int32)
    idx_v[pl.ds(IDX_PAD - 2 * LANES, LANES)] = zeros16
    idx_v[pl.ds(IDX_PAD - LANES, LANES)] = zeros16

    def chunk(ci, carry):
        off, ecnt = carry
        sv = scores_v[pl.ds(ci * LANES, LANES)]
        gt = sv > tvec
        eq = sv == tvec
        eq_i = jnp.where(eq, 1, 0).astype(jnp.int32)
        eq_rank = (ecnt + plsc.cumsum(eq_i)) - eq_i
        keep = gt | (eq & (eq_rank < qvec))
        keep_i = jnp.where(keep, 1, 0).astype(jnp.int32)
        pos = (off + plsc.cumsum(keep_i)) - 1
        idxvec = ci * LANES + lax.iota(jnp.int32, LANES)
        plsc.store_scatter(idx_v, [pos], idxvec, mask=keep)
        return (off + plsc.all_reduce_population_count(keep),
                ecnt + plsc.all_reduce_population_count(eq))

    zsplat = jnp.zeros((LANES,), jnp.int32)
    lax.fori_loop(0, NCHUNK, chunk, (zsplat, zsplat))

    for j in range(2):
        h = r * 2 + j
        bh = b * 8 + h
        for tbl, out in ((keys_hbm, outk_hbm), (values_hbm, outv_hbm)):
            @pl.loop(0, NGF, step=2)
            def _(ci):
                sa = pl.ds(ci * GCH, GCH)
                sb = pl.ds((ci + 1) * GCH, GCH)
                ga = pltpu.async_copy(tbl.at[bh].at[idx_v.at[sa]], bufa_v, gsa)
                gb = pltpu.async_copy(tbl.at[bh].at[idx_v.at[sb]], bufb_v, gsb)
                ga.wait()
                wa = pltpu.async_copy(bufa_v, out.at[bh, sa], wsa)
                gb.wait()
                wb = pltpu.async_copy(bufb_v, out.at[bh, sb], wsb)
                wa.wait()
                wb.wait()

            tsl = pl.ds(NGF * GCH, GCH)
            pltpu.async_copy(tbl.at[bh].at[idx_v.at[tsl]], bufa_v, gsa).wait()
            pltpu.sync_copy(bufa_v.at[pl.ds(0, REM)],
                            out.at[bh, pl.ds(NGF * GCH, REM)])


def _sc_gather(keys3, values3, scores, thr, quota):
    mesh = plsc.VectorSubcoreMesh(core_axis_name="core",
                                  subcore_axis_name="subcore")
    cp = pltpu.CompilerParams()
    if "needs_layout_passes" in pltpu.CompilerParams.__dataclass_fields__:
        cp = dataclasses.replace(cp, needs_layout_passes=False)
    kern = pl.kernel(
        _sc_body,
        compiler_params=cp,
        out_type=(jax.ShapeDtypeStruct((B * H, K, D), jnp.float32),
                  jax.ShapeDtypeStruct((B * H, K, D), jnp.float32)),
        mesh=mesh,
        scratch_types=[
            pltpu.VMEM((S,), jnp.float32),
            pltpu.VMEM((128,), jnp.float32),
            pltpu.VMEM((128,), jnp.int32),
            pltpu.VMEM((IDX_PAD,), jnp.int32),
            pltpu.VMEM((GCH, D), jnp.float32),
            pltpu.VMEM((GCH, D), jnp.float32),
            pltpu.SemaphoreType.DMA,
            pltpu.SemaphoreType.DMA,
            pltpu.SemaphoreType.DMA,
            pltpu.SemaphoreType.DMA,
        ],
    )
    return kern(keys3, values3, scores, thr, quota)


@jax.jit
def kernel(keys, values):
    scores3 = _norms(keys)
    thr, quota = _select(scores3)
    keys3 = keys.reshape(B * H, S, D)
    values3 = values.reshape(B * H, S, D)
    outk, outv = _sc_gather(keys3, values3, scores3,
                            thr.reshape(B, 1, 128), quota.reshape(B, 1, 128))
    return outk.reshape(B, H, K, D), outv.reshape(B, H, K, D)


# speedup vs baseline: 6.1858x; 1.0902x over previous
import dataclasses

import jax
import jax.numpy as jnp
from jax import lax
from jax.experimental import pallas as pl
from jax.experimental.pallas import tpu as pltpu
from jax.experimental.pallas import tpu_sc as plsc

B, H, S, D = 8, 8, 8192, 128
K = max(1, min(S, int(S * (1.0 - 0.8))))
SBLK = 1024
LANES = 16
NCHUNK = S // LANES
GCH = 128
NGF = K // GCH
REM = K - NGF * GCH
IDX_PAD = (NGF + 1) * GCH


def _norms_body(keys_ref, out_ref):
    x = keys_ref[0]
    ss = jnp.sum(x * x, axis=-1)
    out_ref[0, 0] = jnp.mean(jnp.sqrt(ss), axis=0)


def _norms(keys):
    out = pl.pallas_call(
        _norms_body,
        grid=(B, S // SBLK),
        in_specs=[pl.BlockSpec((1, H, SBLK, D), lambda b, s: (b, 0, s, 0))],
        out_specs=pl.BlockSpec((1, 1, SBLK), lambda b, s: (b, 0, s)),
        out_shape=jax.ShapeDtypeStruct((B, 1, S), jnp.float32),
    )(keys)
    return out


def _select_body(scores_ref, thr_ref, quota_ref):
    bits = lax.bitcast_convert_type(scores_ref[:, 0, :], jnp.int32)

    def body(_, carry):
        lo, hi = carry
        mid = lo + ((hi - lo + 1) >> 1)
        cnt = jnp.sum((bits >= mid).astype(jnp.int32), axis=1, keepdims=True)
        ge = cnt >= K
        return jnp.where(ge, mid, lo), jnp.where(ge, hi, mid - 1)

    lo0 = jnp.zeros((B, 1), jnp.int32)
    hi0 = jnp.full((B, 1), 0x7F800000, jnp.int32)
    lo, _ = lax.fori_loop(0, 31, body, (lo0, hi0))
    cnt_gt = jnp.sum((bits >= (lo + 1)).astype(jnp.int32), axis=1,
                     keepdims=True)
    quota = K - cnt_gt
    thr_ref[...] = jnp.broadcast_to(
        lax.bitcast_convert_type(lo, jnp.float32), (B, 128))
    quota_ref[...] = jnp.broadcast_to(quota, (B, 128))


def _select(scores):
    return pl.pallas_call(
        _select_body,
        grid=(1,),
        in_specs=[pl.BlockSpec((B, 1, S), lambda i: (0, 0, 0))],
        out_specs=(pl.BlockSpec((B, 128), lambda i: (0, 0)),
                   pl.BlockSpec((B, 128), lambda i: (0, 0))),
        out_shape=(jax.ShapeDtypeStruct((B, 128), jnp.float32),
                   jax.ShapeDtypeStruct((B, 128), jnp.int32)),
    )(scores)


def _sc_body(keys_hbm, values_hbm, scores_hbm, thr_hbm, quota_hbm,
             outk_hbm, outv_hbm,
             scores_v, thr_v, quota_v, idx_v, bufa_v, bufb_v,
             gsa, gsb, wsa, wsb):
    c = lax.axis_index("core")
    sub = lax.axis_index("subcore")
    wid = sub * 2 + c
    b = wid // 4
    r = wid % 4

    pltpu.sync_copy(scores_hbm.at[b, 0], scores_v)
    pltpu.sync_copy(thr_hbm.at[b, 0], thr_v)
    pltpu.sync_copy(quota_hbm.at[b, 0], quota_v)
    tvec = thr_v[pl.ds(0, LANES)]
    qvec = quota_v[pl.ds(0, LANES)]

    zeros16 = jnp.zeros((LANES,), jnp.int32)
    idx_v[pl.ds(IDX_PAD - 2 * LANES, LANES)] = zeros16
    idx_v[pl.ds(IDX_PAD - LANES, LANES)] = zeros16

    def chunk(ci, carry):
        off, ecnt = carry
        sv = scores_v[pl.ds(ci * LANES, LANES)]
        gt = sv > tvec
        eq = sv == tvec
        eq_i = jnp.where(eq, 1, 0).astype(jnp.int32)
        eq_rank = (ecnt + plsc.cumsum(eq_i)) - eq_i
        keep = gt | (eq & (eq_rank < qvec))
        keep_i = jnp.where(keep, 1, 0).astype(jnp.int32)
        pos = (off + plsc.cumsum(keep_i)) - 1
        idxvec = ci * LANES + lax.iota(jnp.int32, LANES)
        plsc.store_scatter(idx_v, [pos], idxvec, mask=keep)
        return (off + plsc.all_reduce_population_count(keep),
                ecnt + plsc.all_reduce_population_count(eq))

    zsplat = jnp.zeros((LANES,), jnp.int32)
    lax.fori_loop(0, NCHUNK, chunk, (zsplat, zsplat))

    for j in range(2):
        h = r * 2 + j
        bh = b * 8 + h
        for tbl, out in ((keys_hbm, outk_hbm), (values_hbm, outv_hbm)):
            @pl.loop(0, NGF, step=2)
            def _(ci):
                sa = pl.ds(ci * GCH, GCH)
                sb = pl.ds((ci + 1) * GCH, GCH)
                ga = pltpu.async_copy(tbl.at[bh].at[idx_v.at[sa]], bufa_v, gsa)
                gb = pltpu.async_copy(tbl.at[bh].at[idx_v.at[sb]], bufb_v, gsb)
                ga.wait()
                wa = pltpu.async_copy(bufa_v, out.at[b, h, sa], wsa)
                gb.wait()
                wb = pltpu.async_copy(bufb_v, out.at[b, h, sb], wsb)
                wa.wait()
                wb.wait()

            tsl = pl.ds(NGF * GCH, GCH)
            pltpu.async_copy(tbl.at[bh].at[idx_v.at[tsl]], bufa_v, gsa).wait()
            pltpu.sync_copy(bufa_v.at[pl.ds(0, REM)],
                            out.at[b, h, pl.ds(NGF * GCH, REM)])


def _sc_gather(keys3, values3, scores, thr, quota):
    mesh = plsc.VectorSubcoreMesh(core_axis_name="core",
                                  subcore_axis_name="subcore")
    cp = pltpu.CompilerParams()
    if "needs_layout_passes" in pltpu.CompilerParams.__dataclass_fields__:
        cp = dataclasses.replace(cp, needs_layout_passes=False)
    kern = pl.kernel(
        _sc_body,
        compiler_params=cp,
        out_type=(jax.ShapeDtypeStruct((B, H, K, D), jnp.float32),
                  jax.ShapeDtypeStruct((B, H, K, D), jnp.float32)),
        mesh=mesh,
        scratch_types=[
            pltpu.VMEM((S,), jnp.float32),
            pltpu.VMEM((128,), jnp.float32),
            pltpu.VMEM((128,), jnp.int32),
            pltpu.VMEM((IDX_PAD,), jnp.int32),
            pltpu.VMEM((GCH, D), jnp.float32),
            pltpu.VMEM((GCH, D), jnp.float32),
            pltpu.SemaphoreType.DMA,
            pltpu.SemaphoreType.DMA,
            pltpu.SemaphoreType.DMA,
            pltpu.SemaphoreType.DMA,
        ],
    )
    return kern(keys3, values3, scores, thr, quota)


@jax.jit
def kernel(keys, values):
    scores3 = _norms(keys)
    thr, quota = _select(scores3)
    keys3 = keys.reshape(B * H, S, D)
    values3 = values.reshape(B * H, S, D)
    outk, outv = _sc_gather(keys3, values3, scores3,
                            thr.reshape(B, 1, 128), quota.reshape(B, 1, 128))
    return outk, outv


# speedup vs baseline: 6.3779x; 1.0311x over previous
import dataclasses

import jax
import jax.numpy as jnp
from jax import lax
from jax.experimental import pallas as pl
from jax.experimental.pallas import tpu as pltpu
from jax.experimental.pallas import tpu_sc as plsc

B, H, S, D = 8, 8, 8192, 128
K = max(1, min(S, int(S * (1.0 - 0.8))))
SBLK = 1024
LANES = 16
NCHUNK = S // LANES
GCH = 128
NGF = K // GCH
REM = K - NGF * GCH
IDX_PAD = (NGF + 1) * GCH


def _norms_body(keys_ref, out_ref):
    x = keys_ref[0].reshape(H * SBLK, D)
    x2t = lax.transpose(x * x, (1, 0))
    ss = jnp.sum(x2t, axis=0)
    nrm = jnp.sqrt(ss)
    acc = nrm[0:SBLK]
    for hh in range(1, H):
        acc = acc + nrm[hh * SBLK:(hh + 1) * SBLK]
    out_ref[0, 0] = acc * (1.0 / H)


def _norms(keys):
    out = pl.pallas_call(
        _norms_body,
        grid=(B, S // SBLK),
        in_specs=[pl.BlockSpec((1, H, SBLK, D), lambda b, s: (b, 0, s, 0))],
        out_specs=pl.BlockSpec((1, 1, SBLK), lambda b, s: (b, 0, s)),
        out_shape=jax.ShapeDtypeStruct((B, 1, S), jnp.float32),
    )(keys)
    return out


def _select_body(scores_ref, thr_ref, quota_ref):
    bits = lax.bitcast_convert_type(scores_ref[:, 0, :], jnp.int32)

    def body(_, carry):
        lo, hi = carry
        mid = lo + ((hi - lo + 1) >> 1)
        cnt = jnp.sum((bits >= mid).astype(jnp.int32), axis=1, keepdims=True)
        ge = cnt >= K
        return jnp.where(ge, mid, lo), jnp.where(ge, hi, mid - 1)

    lo0 = jnp.zeros((B, 1), jnp.int32)
    hi0 = jnp.full((B, 1), 0x7F800000, jnp.int32)
    lo, _ = lax.fori_loop(0, 31, body, (lo0, hi0))
    cnt_gt = jnp.sum((bits >= (lo + 1)).astype(jnp.int32), axis=1,
                     keepdims=True)
    quota = K - cnt_gt
    thr_ref[...] = jnp.broadcast_to(
        lax.bitcast_convert_type(lo, jnp.float32), (B, 128))
    quota_ref[...] = jnp.broadcast_to(quota, (B, 128))


def _select(scores):
    return pl.pallas_call(
        _select_body,
        grid=(1,),
        in_specs=[pl.BlockSpec((B, 1, S), lambda i: (0, 0, 0))],
        out_specs=(pl.BlockSpec((B, 128), lambda i: (0, 0)),
                   pl.BlockSpec((B, 128), lambda i: (0, 0))),
        out_shape=(jax.ShapeDtypeStruct((B, 128), jnp.float32),
                   jax.ShapeDtypeStruct((B, 128), jnp.int32)),
    )(scores)


def _sc_body(keys_hbm, values_hbm, scores_hbm, thr_hbm, quota_hbm,
             outk_hbm, outv_hbm,
             scores_v, thr_v, quota_v, idx_v, bufa_v, bufb_v,
             gsa, gsb, wsa, wsb):
    c = lax.axis_index("core")
    sub = lax.axis_index("subcore")
    wid = sub * 2 + c
    b = wid // 4
    r = wid % 4

    pltpu.sync_copy(scores_hbm.at[b, 0], scores_v)
    pltpu.sync_copy(thr_hbm.at[b, 0], thr_v)
    pltpu.sync_copy(quota_hbm.at[b, 0], quota_v)
    tvec = thr_v[pl.ds(0, LANES)]
    qvec = quota_v[pl.ds(0, LANES)]

    zeros16 = jnp.zeros((LANES,), jnp.int32)
    idx_v[pl.ds(IDX_PAD - 2 * LANES, LANES)] = zeros16
    idx_v[pl.ds(IDX_PAD - LANES, LANES)] = zeros16

    def chunk(ci, carry):
        off, ecnt = carry
        sv = scores_v[pl.ds(ci * LANES, LANES)]
        gt = sv > tvec
        eq = sv == tvec
        eq_i = jnp.where(eq, 1, 0).astype(jnp.int32)
        eq_rank = (ecnt + plsc.cumsum(eq_i)) - eq_i
        keep = gt | (eq & (eq_rank < qvec))
        keep_i = jnp.where(keep, 1, 0).astype(jnp.int32)
        pos = (off + plsc.cumsum(keep_i)) - 1
        idxvec = ci * LANES + lax.iota(jnp.int32, LANES)
        plsc.store_scatter(idx_v, [pos], idxvec, mask=keep)
        return (off + plsc.all_reduce_population_count(keep),
                ecnt + plsc.all_reduce_population_count(eq))

    zsplat = jnp.zeros((LANES,), jnp.int32)
    lax.fori_loop(0, NCHUNK, chunk, (zsplat, zsplat))

    for j in range(2):
        h = r * 2 + j
        bh = b * 8 + h
        for tbl, out in ((keys_hbm, outk_hbm), (values_hbm, outv_hbm)):
            @pl.loop(0, NGF, step=2)
            def _(ci):
                sa = pl.ds(ci * GCH, GCH)
                sb = pl.ds((ci + 1) * GCH, GCH)
                ga = pltpu.async_copy(tbl.at[bh].at[idx_v.at[sa]], bufa_v, gsa)
                gb = pltpu.async_copy(tbl.at[bh].at[idx_v.at[sb]], bufb_v, gsb)
                ga.wait()
                wa = pltpu.async_copy(bufa_v, out.at[b, h, sa], wsa)
                gb.wait()
                wb = pltpu.async_copy(bufb_v, out.at[b, h, sb], wsb)
                wa.wait()
                wb.wait()

            tsl = pl.ds(NGF * GCH, GCH)
            pltpu.async_copy(tbl.at[bh].at[idx_v.at[tsl]], bufa_v, gsa).wait()
            pltpu.sync_copy(bufa_v.at[pl.ds(0, REM)],
                            out.at[b, h, pl.ds(NGF * GCH, REM)])


def _sc_gather(keys3, values3, scores, thr, quota):
    mesh = plsc.VectorSubcoreMesh(core_axis_name="core",
                                  subcore_axis_name="subcore")
    cp = pltpu.CompilerParams()
    if "needs_layout_passes" in pltpu.CompilerParams.__dataclass_fields__:
        cp = dataclasses.replace(cp, needs_layout_passes=False)
    kern = pl.kernel(
        _sc_body,
        compiler_params=cp,
        out_type=(jax.ShapeDtypeStruct((B, H, K, D), jnp.float32),
                  jax.ShapeDtypeStruct((B, H, K, D), jnp.float32)),
        mesh=mesh,
        scratch_types=[
            pltpu.VMEM((S,), jnp.float32),
            pltpu.VMEM((128,), jnp.float32),
            pltpu.VMEM((128,), jnp.int32),
            pltpu.VMEM((IDX_PAD,), jnp.int32),
            pltpu.VMEM((GCH, D), jnp.float32),
            pltpu.VMEM((GCH, D), jnp.float32),
            pltpu.SemaphoreType.DMA,
            pltpu.SemaphoreType.DMA,
            pltpu.SemaphoreType.DMA,
            pltpu.SemaphoreType.DMA,
        ],
    )
    return kern(keys3, values3, scores, thr, quota)


@jax.jit
def kernel(keys, values):
    scores3 = _norms(keys)
    thr, quota = _select(scores3)
    keys3 = keys.reshape(B * H, S, D)
    values3 = values.reshape(B * H, S, D)
    outk, outv = _sc_gather(keys3, values3, scores3,
                            thr.reshape(B, 1, 128), quota.reshape(B, 1, 128))
    return outk, outv


# speedup vs baseline: 8.5731x; 1.3442x over previous
import dataclasses

import jax
import jax.numpy as jnp
from jax import lax
from jax.experimental import pallas as pl
from jax.experimental.pallas import tpu as pltpu
from jax.experimental.pallas import tpu_sc as plsc

B, H, S, D = 8, 8, 8192, 128
K = max(1, min(S, int(S * (1.0 - 0.8))))
SBLK = 1024
LANES = 16
NCHUNK = S // LANES
NKC = (K + 15) // 16
KCQ = 26
IDX_PAD = NKC * LANES
GIW = KCQ * LANES * H


def _norms_body(keys_ref, out_ref):
    x = keys_ref[0].reshape(H * SBLK, D)
    x2t = lax.transpose(x * x, (1, 0))
    ss = jnp.sum(x2t, axis=0)
    nrm = jnp.sqrt(ss)
    acc = nrm[0:SBLK]
    for hh in range(1, H):
        acc = acc + nrm[hh * SBLK:(hh + 1) * SBLK]
    out_ref[0, 0] = acc * (1.0 / H)


def _norms(keys):
    out = pl.pallas_call(
        _norms_body,
        grid=(B, S // SBLK),
        in_specs=[pl.BlockSpec((1, H, SBLK, D), lambda b, s: (b, 0, s, 0))],
        out_specs=pl.BlockSpec((1, 1, SBLK), lambda b, s: (b, 0, s)),
        out_shape=jax.ShapeDtypeStruct((B, 1, S), jnp.float32),
    )(keys)
    return out


def _select_body(scores_ref, thr_ref, quota_ref):
    bits = lax.bitcast_convert_type(scores_ref[:, 0, :], jnp.int32)

    def body(_, carry):
        lo, hi = carry
        mid = lo + ((hi - lo + 1) >> 1)
        cnt = jnp.sum((bits >= mid).astype(jnp.int32), axis=1, keepdims=True)
        ge = cnt >= K
        return jnp.where(ge, mid, lo), jnp.where(ge, hi, mid - 1)

    lo0 = jnp.zeros((B, 1), jnp.int32)
    hi0 = jnp.full((B, 1), 0x7F800000, jnp.int32)
    lo, _ = lax.fori_loop(0, 31, body, (lo0, hi0))
    cnt_gt = jnp.sum((bits >= (lo + 1)).astype(jnp.int32), axis=1,
                     keepdims=True)
    quota = K - cnt_gt
    thr_ref[...] = jnp.broadcast_to(
        lax.bitcast_convert_type(lo, jnp.float32), (B, 128))
    quota_ref[...] = jnp.broadcast_to(quota, (B, 128))


def _select(scores):
    return pl.pallas_call(
        _select_body,
        grid=(1,),
        in_specs=[pl.BlockSpec((B, 1, S), lambda i: (0, 0, 0))],
        out_specs=(pl.BlockSpec((B, 128), lambda i: (0, 0)),
                   pl.BlockSpec((B, 128), lambda i: (0, 0))),
        out_shape=(jax.ShapeDtypeStruct((B, 128), jnp.float32),
                   jax.ShapeDtypeStruct((B, 128), jnp.int32)),
    )(scores)


def _sc_body(keys_hbm, values_hbm, scores_hbm, thr_hbm, quota_hbm,
             outk_hbm, outv_hbm,
             scores_v, thr_v, quota_v, idx_v, gidx_v, bufa_v, bufb_v,
             gsa, gsb, wsa, wsb):
    c = lax.axis_index("core")
    sub = lax.axis_index("subcore")
    wid = sub * 2 + c
    b = wid // 4
    q = wid % 4

    pltpu.sync_copy(scores_hbm.at[b, 0], scores_v)
    pltpu.sync_copy(thr_hbm.at[b, 0], thr_v)
    pltpu.sync_copy(quota_hbm.at[b, 0], quota_v)
    tvec = thr_v[pl.ds(0, LANES)]
    qvec = quota_v[pl.ds(0, LANES)]

    def chunk(ci, carry):
        off, ecnt = carry
        sv = scores_v[pl.ds(ci * LANES, LANES)]
        gt = sv > tvec
        eq = sv == tvec
        eq_i = jnp.where(eq, 1, 0).astype(jnp.int32)
        eq_rank = (ecnt + plsc.cumsum(eq_i)) - eq_i
        keep = gt | (eq & (eq_rank < qvec))
        keep_i = jnp.where(keep, 1, 0).astype(jnp.int32)
        pos = (off + plsc.cumsum(keep_i)) - 1
        idxvec = ci * LANES + lax.iota(jnp.int32, LANES)
        plsc.store_scatter(idx_v, [pos], idxvec, mask=keep)
        return (off + plsc.all_reduce_population_count(keep),
                ecnt + plsc.all_reduce_population_count(eq))

    zsplat = jnp.zeros((LANES,), jnp.int32)
    lax.fori_loop(0, NCHUNK, chunk, (zsplat, zsplat))

    c0 = q * KCQ
    c1 = jnp.minimum((q + 1) * KCQ, NKC)
    c1f = jnp.minimum(c1, NKC - 1)

    lanes = lax.iota(jnp.int32, LANES)
    base = b * (H * S)

    def bchunk(ci, _):
        lc = ci - c0
        iv = idx_v[pl.ds(ci * LANES, LANES)]
        valid = (ci * LANES + lanes) < K
        for hh in range(H):
            vals = iv + (base + hh * S)
            posv = (lc * LANES + lanes) * H + hh
            plsc.store_scatter(gidx_v, [posv], vals, mask=valid)
        return 0

    lax.fori_loop(c0, c1, bchunk, 0)

    for tbl, out in ((keys_hbm, outk_hbm), (values_hbm, outv_hbm)):
        npairs = (c1f - c0) // 2

        def gpair(i, _):
            ca = c0 + 2 * i
            la = 2 * i * 128
            sa = pl.ds(la, 128)
            sb = pl.ds(la + 128, 128)
            ga = pltpu.async_copy(tbl.at[gidx_v.at[sa]], bufa_v, gsa)
            gb = pltpu.async_copy(tbl.at[gidx_v.at[sb]], bufb_v, gsb)
            ga.wait()
            wa = pltpu.async_copy(bufa_v, out.at[b, pl.ds(ca * 128, 128)], wsa)
            gb.wait()
            wb = pltpu.async_copy(bufb_v, out.at[b, pl.ds(ca * 128 + 128, 128)],
                                  wsb)
            wa.wait()
            wb.wait()
            return 0

        lax.fori_loop(0, npairs, gpair, 0)

        @pl.when(c1 == NKC)
        def _():
            rem = (K - (NKC - 1) * LANES) * H
            lt = (NKC - 1 - c0) * 128
            g = pltpu.async_copy(tbl.at[gidx_v.at[pl.ds(lt, rem)]],
                                 bufa_v.at[pl.ds(0, rem)], gsa)
            g.wait()
            pltpu.sync_copy(bufa_v.at[pl.ds(0, rem)],
                            out.at[b, pl.ds((NKC - 1) * 128, rem)])


def _sc_gather(keys2, values2, scores, thr, quota):
    mesh = plsc.VectorSubcoreMesh(core_axis_name="core",
                                  subcore_axis_name="subcore")
    cp = pltpu.CompilerParams()
    if "needs_layout_passes" in pltpu.CompilerParams.__dataclass_fields__:
        cp = dataclasses.replace(cp, needs_layout_passes=False)
    kern = pl.kernel(
        _sc_body,
        compiler_params=cp,
        out_type=(jax.ShapeDtypeStruct((B, K * H, D), jnp.float32),
                  jax.ShapeDtypeStruct((B, K * H, D), jnp.float32)),
        mesh=mesh,
        scratch_types=[
            pltpu.VMEM((S,), jnp.float32),
            pltpu.VMEM((128,), jnp.float32),
            pltpu.VMEM((128,), jnp.int32),
            pltpu.VMEM((IDX_PAD,), jnp.int32),
            pltpu.VMEM((GIW,), jnp.int32),
            pltpu.VMEM((128, D), jnp.float32),
            pltpu.VMEM((128, D), jnp.float32),
            pltpu.SemaphoreType.DMA,
            pltpu.SemaphoreType.DMA,
            pltpu.SemaphoreType.DMA,
            pltpu.SemaphoreType.DMA,
        ],
    )
    return kern(keys2, values2, scores, thr, quota)


@jax.jit
def kernel(keys, values):
    scores3 = _norms(keys)
    thr, quota = _select(scores3)
    keys2 = keys.reshape(B * H * S, D)
    values2 = values.reshape(B * H * S, D)
    outk3, outv3 = _sc_gather(keys2, values2, scores3,
                              thr.reshape(B, 1, 128), quota.reshape(B, 1, 128))
    outk = outk3.reshape(B, K, H, D).transpose(0, 2, 1, 3)
    outv = outv3.reshape(B, K, H, D).transpose(0, 2, 1, 3)
    return outk, outv


# speedup vs baseline: 8.6675x; 1.0110x over previous
import dataclasses

import jax
import jax.numpy as jnp
from jax import lax
from jax.experimental import pallas as pl
from jax.experimental.pallas import tpu as pltpu
from jax.experimental.pallas import tpu_sc as plsc

B, H, S, D = 8, 8, 8192, 128
K = max(1, min(S, int(S * (1.0 - 0.8))))
SBLK = 1024
LANES = 16
NCHUNK = S // LANES
NKC = (K + 15) // 16
KCQ = 26
IDX_PAD = NKC * LANES
GIW = KCQ * LANES * H


def _norms_body(keys_ref, out_ref):
    x = keys_ref[0].reshape(H * SBLK, D)
    x2t = lax.transpose(x * x, (1, 0))
    ss = jnp.sum(x2t, axis=0)
    nrm = jnp.sqrt(ss)
    acc = nrm[0:SBLK]
    for hh in range(1, H):
        acc = acc + nrm[hh * SBLK:(hh + 1) * SBLK]
    out_ref[0, 0] = acc * (1.0 / H)


def _norms(keys):
    out = pl.pallas_call(
        _norms_body,
        grid=(B, S // SBLK),
        in_specs=[pl.BlockSpec((1, H, SBLK, D), lambda b, s: (b, 0, s, 0))],
        out_specs=pl.BlockSpec((1, 1, SBLK), lambda b, s: (b, 0, s)),
        out_shape=jax.ShapeDtypeStruct((B, 1, S), jnp.float32),
    )(keys)
    return out


def _select_body(scores_ref, thr_ref, quota_ref):
    bits = lax.bitcast_convert_type(scores_ref[:, 0, :], jnp.int32)

    def body(_, carry):
        lo, hi = carry
        mid = lo + ((hi - lo + 1) >> 1)
        cnt = jnp.sum((bits >= mid).astype(jnp.int32), axis=1, keepdims=True)
        ge = cnt >= K
        return jnp.where(ge, mid, lo), jnp.where(ge, hi, mid - 1)

    lo0 = jnp.zeros((B, 1), jnp.int32)
    hi0 = jnp.full((B, 1), 0x7F800000, jnp.int32)
    lo, _ = lax.fori_loop(0, 31, body, (lo0, hi0))
    cnt_gt = jnp.sum((bits >= (lo + 1)).astype(jnp.int32), axis=1,
                     keepdims=True)
    quota = K - cnt_gt
    thr_ref[...] = jnp.broadcast_to(
        lax.bitcast_convert_type(lo, jnp.float32), (B, 128))
    quota_ref[...] = jnp.broadcast_to(quota, (B, 128))


def _select(scores):
    return pl.pallas_call(
        _select_body,
        grid=(1,),
        in_specs=[pl.BlockSpec((B, 1, S), lambda i: (0, 0, 0))],
        out_specs=(pl.BlockSpec((B, 128), lambda i: (0, 0)),
                   pl.BlockSpec((B, 128), lambda i: (0, 0))),
        out_shape=(jax.ShapeDtypeStruct((B, 128), jnp.float32),
                   jax.ShapeDtypeStruct((B, 128), jnp.int32)),
    )(scores)


def _sc_body(keys_hbm, values_hbm, scores_hbm, thr_hbm, quota_hbm,
             outk_hbm, outv_hbm,
             scores_v, thr_v, quota_v, idx_v, gidx_v, bufa_v, bufb_v,
             gsa, gsb, wsa, wsb):
    c = lax.axis_index("core")
    sub = lax.axis_index("subcore")
    wid = sub * 2 + c
    b = wid // 4
    q = wid % 4

    pltpu.sync_copy(scores_hbm.at[b, 0], scores_v)
    pltpu.sync_copy(thr_hbm.at[b, 0], thr_v)
    pltpu.sync_copy(quota_hbm.at[b, 0], quota_v)
    tvec = thr_v[pl.ds(0, LANES)]
    qvec = quota_v[pl.ds(0, LANES)]

    zsplat = jnp.zeros((LANES,), jnp.int32)
    iota16 = lax.iota(jnp.int32, LANES)

    def chunk_ge(ci, off):
        sv = scores_v[pl.ds(ci * LANES, LANES)]
        ge = sv >= tvec
        ge_i = jnp.where(ge, 1, 0).astype(jnp.int32)
        pos = jnp.minimum((off + plsc.cumsum(ge_i)) - 1, IDX_PAD - 1)
        plsc.store_scatter(idx_v, [pos], ci * LANES + iota16, mask=ge)
        return off + plsc.all_reduce_population_count(ge)

    total = lax.fori_loop(0, NCHUNK, chunk_ge, zsplat)

    @pl.when(jnp.max(total) > K)
    def _():
        def chunk(ci, carry):
            off, ecnt = carry
            sv = scores_v[pl.ds(ci * LANES, LANES)]
            gt = sv > tvec
            eq = sv == tvec
            eq_i = jnp.where(eq, 1, 0).astype(jnp.int32)
            eq_rank = (ecnt + plsc.cumsum(eq_i)) - eq_i
            keep = gt | (eq & (eq_rank < qvec))
            keep_i = jnp.where(keep, 1, 0).astype(jnp.int32)
            pos = (off + plsc.cumsum(keep_i)) - 1
            plsc.store_scatter(idx_v, [pos], ci * LANES + iota16, mask=keep)
            return (off + plsc.all_reduce_population_count(keep),
                    ecnt + plsc.all_reduce_population_count(eq))

        lax.fori_loop(0, NCHUNK, chunk, (zsplat, zsplat))

    c0 = q * KCQ
    c1 = jnp.minimum((q + 1) * KCQ, NKC)
    c1f = jnp.minimum(c1, NKC - 1)

    lanes = lax.iota(jnp.int32, LANES)
    base = b * (H * S)

    def bchunk(ci, _):
        lc = ci - c0
        iv = idx_v[pl.ds(ci * LANES, LANES)]
        valid = (ci * LANES + lanes) < K
        for hh in range(H):
            vals = iv + (base + hh * S)
            posv = (lc * LANES + lanes) * H + hh
            plsc.store_scatter(gidx_v, [posv], vals, mask=valid)
        return 0

    lax.fori_loop(c0, c1, bchunk, 0)

    for tbl, out in ((keys_hbm, outk_hbm), (values_hbm, outv_hbm)):
        npairs = (c1f - c0) // 2

        def gpair(i, _):
            ca = c0 + 2 * i
            la = 2 * i * 128
            sa = pl.ds(la, 128)
            sb = pl.ds(la + 128, 128)
            ga = pltpu.async_copy(tbl.at[gidx_v.at[sa]], bufa_v, gsa)
            gb = pltpu.async_copy(tbl.at[gidx_v.at[sb]], bufb_v, gsb)
            ga.wait()
            wa = pltpu.async_copy(bufa_v, out.at[b, pl.ds(ca * 128, 128)], wsa)
            gb.wait()
            wb = pltpu.async_copy(bufb_v, out.at[b, pl.ds(ca * 128 + 128, 128)],
                                  wsb)
            wa.wait()
            wb.wait()
            return 0

        lax.fori_loop(0, npairs, gpair, 0)

        @pl.when(c1 == NKC)
        def _():
            rem = (K - (NKC - 1) * LANES) * H
            lt = (NKC - 1 - c0) * 128
            g = pltpu.async_copy(tbl.at[gidx_v.at[pl.ds(lt, rem)]],
                                 bufa_v.at[pl.ds(0, rem)], gsa)
            g.wait()
            pltpu.sync_copy(bufa_v.at[pl.ds(0, rem)],
                            out.at[b, pl.ds((NKC - 1) * 128, rem)])


def _sc_gather(keys2, values2, scores, thr, quota):
    mesh = plsc.VectorSubcoreMesh(core_axis_name="core",
                                  subcore_axis_name="subcore")
    cp = pltpu.CompilerParams()
    if "needs_layout_passes" in pltpu.CompilerParams.__dataclass_fields__:
        cp = dataclasses.replace(cp, needs_layout_passes=False)
    kern = pl.kernel(
        _sc_body,
        compiler_params=cp,
        out_type=(jax.ShapeDtypeStruct((B, K * H, D), jnp.float32),
                  jax.ShapeDtypeStruct((B, K * H, D), jnp.float32)),
        mesh=mesh,
        scratch_types=[
            pltpu.VMEM((S,), jnp.float32),
            pltpu.VMEM((128,), jnp.float32),
            pltpu.VMEM((128,), jnp.int32),
            pltpu.VMEM((IDX_PAD,), jnp.int32),
            pltpu.VMEM((GIW,), jnp.int32),
            pltpu.VMEM((128, D), jnp.float32),
            pltpu.VMEM((128, D), jnp.float32),
            pltpu.SemaphoreType.DMA,
            pltpu.SemaphoreType.DMA,
            pltpu.SemaphoreType.DMA,
            pltpu.SemaphoreType.DMA,
        ],
    )
    return kern(keys2, values2, scores, thr, quota)


@jax.jit
def kernel(keys, values):
    scores3 = _norms(keys)
    thr, quota = _select(scores3)
    keys2 = keys.reshape(B * H * S, D)
    values2 = values.reshape(B * H * S, D)
    outk3, outv3 = _sc_gather(keys2, values2, scores3,
                              thr.reshape(B, 1, 128), quota.reshape(B, 1, 128))
    outk = outk3.reshape(B, K, H, D).transpose(0, 2, 1, 3)
    outv = outv3.reshape(B, K, H, D).transpose(0, 2, 1, 3)
    return outk, outv
